# baseline (speedup 1.0000x reference)
import functools

import jax
import jax.numpy as jnp
from jax import lax
from jax.experimental import pallas as pl
from jax.experimental.pallas import tpu as pltpu
from jax.experimental.pallas import tpu_sc as plsc

N = 10000
E = 320000
D_IN = 128
H = 64
L = 4
CHUNK = 128
NB = 4
DEG_W = 16
BN_EPS = 1e-5
CORE0_SHARE = 1.0


def _geom(nc, ns):
    nw = nc * ns
    n_chunks = -(-E // CHUNK)
    ch = [0] * nc
    ch[0] = max(NB, round(n_chunks * CORE0_SHARE / ns / NB) * NB)
    if nc > 1:
        rest = max(0, n_chunks - ns * ch[0])
        per = -(-rest // ((nc - 1) * ns))
        cho = max(NB, -(-per // NB) * NB)
        for c in range(1, nc):
            ch[c] = cho
    tot_ch = ns * sum(ch)
    tot_ch_pad = tot_ch + max(ch)
    ch_deg = -(-tot_ch // nw)
    tot_ch_pad = max(tot_ch_pad, nw * ch_deg)
    n_pad = ns * CHUNK * (-(-(N + 1) // (ns * CHUNK)))
    return nw, tuple(ch), ch_deg, tot_ch, tot_ch_pad, n_pad


@functools.cache
def _sc_calls(nc, ns):
    nw, ch, ch_deg, tot_ch, tot_ch_pad, n_pad = _geom(nc, ns)
    ch_max = max(ch)
    rpt = n_pad // ns
    nzc = rpt // CHUNK
    mesh = plsc.VectorSubcoreMesh(
        core_axis_name="c", subcore_axis_name="s",
        num_cores=nc, num_subcores=ns)
    cparams = pltpu.CompilerParams(use_tc_tiling_on_sc=False)

    @functools.partial(
        pl.kernel,
        out_type=jax.ShapeDtypeStruct((nc, n_pad, DEG_W), jnp.float32),
        mesh=mesh,
        compiler_params=cparams,
        scratch_types=[
            pltpu.VMEM((ch_deg, CHUNK), jnp.int32),
            pltpu.VMEM((CHUNK, DEG_W), jnp.float32),
            pltpu.VMEM_SHARED((n_pad, DEG_W), jnp.float32),
        ],
    )
    def degree(dst_hbm, out_hbm, dst_v, val_v, acc):
        cid = lax.axis_index("c")
        sid = lax.axis_index("s")
        wid = sid * nc + cid
        pltpu.sync_copy(dst_hbm.at[pl.ds(wid * ch_deg, ch_deg)], dst_v)

        def _fill(c):
            v = jnp.full((16,), c, jnp.float32)

            def _f(i, _):
                val_v[i, pl.ds(0, 16)] = v
                return 0

            lax.fori_loop(0, CHUNK, _f, 0)

        _fill(0.0)
        for q in range(nzc):
            pltpu.sync_copy(val_v, acc.at[pl.ds(sid * rpt + q * CHUNK, CHUNK)])
        plsc.subcore_barrier()

        _fill(1.0)

        def _step(j, _):
            pltpu.sync_copy(val_v, acc.at[dst_v.at[j]], add=True)
            return 0

        lax.fori_loop(0, ch_deg, _step, 0)
        plsc.subcore_barrier()

        for q in range(nzc):
            r0 = sid * rpt + q * CHUNK
            pltpu.sync_copy(acc.at[pl.ds(r0, CHUNK)], val_v)
            pltpu.sync_copy(val_v, out_hbm.at[cid, pl.ds(r0, CHUNK)])

    @functools.partial(
        pl.kernel,
        out_type=jax.ShapeDtypeStruct((nc, n_pad, H), jnp.float32),
        mesh=mesh,
        compiler_params=cparams,
        scratch_types=[
            pltpu.VMEM((ch_max, CHUNK), jnp.int32),
            pltpu.VMEM((ch_max, CHUNK), jnp.int32),
            pltpu.VMEM((NB, CHUNK, H), jnp.float32),
            pltpu.VMEM_SHARED((n_pad, H), jnp.float32),
            pltpu.SemaphoreType.DMA,
        ],
    )
    def edge_agg(src_hbm, dst_hbm, u_hbm, out_hbm, src_v, dst_v, ring, acc, sem):
        cid = lax.axis_index("c")
        sid = lax.axis_index("s")
        core_base = 0
        ch_w = jnp.int32(ch[0])
        base_ch = sid * ch[0]
        for c in range(1, nc):
            core_base += ns * ch[c - 1]
            ch_w = jnp.where(cid == c, jnp.int32(ch[c]), ch_w)
            base_ch = jnp.where(cid == c, core_base + sid * ch[c], base_ch)
        with jax.named_scope("stage_idx"):
            for c in range(nc):
                if ch[c] > 0:
                    @pl.when(cid == c)
                    def _():
                        pltpu.sync_copy(src_hbm.at[pl.ds(base_ch, ch[c])],
                                        src_v.at[pl.ds(0, ch[c])])
                        pltpu.sync_copy(dst_hbm.at[pl.ds(base_ch, ch[c])],
                                        dst_v.at[pl.ds(0, ch[c])])

        with jax.named_scope("zero_acc"):
            zero = jnp.zeros((16,), jnp.float32)

            def _zf(t, _):
                i = t // (H // 16)
                k = t % (H // 16)
                ring[0, i, pl.ds(k * 16, 16)] = zero
                return 0

            lax.fori_loop(0, CHUNK * (H // 16), _zf, 0)
            for q in range(nzc):
                pltpu.sync_copy(ring.at[0],
                                acc.at[pl.ds(sid * rpt + q * CHUNK, CHUNK)])
            plsc.subcore_barrier()

        with jax.named_scope("edges"):
            for b in range(NB):
                pltpu.async_copy(u_hbm.at[src_v.at[b]], ring.at[b], sem)

            def _step(g, _):
                for b in range(NB):
                    j = g * NB + b
                    pltpu.make_async_copy(
                        u_hbm.at[pl.ds(0, CHUNK)], ring.at[b], sem).wait()
                    pltpu.sync_copy(ring.at[b], acc.at[dst_v.at[j]], add=True)

                    @pl.when(j + NB < ch_w)
                    def _():
                        pltpu.async_copy(
                            u_hbm.at[src_v.at[j + NB]], ring.at[b], sem)

                return 0

            lax.fori_loop(0, ch_w // NB, _step, 0)
            plsc.subcore_barrier()

        with jax.named_scope("readout"):
            for q in range(nzc):
                r0 = sid * rpt + q * CHUNK
                pltpu.sync_copy(acc.at[pl.ds(r0, CHUNK)], ring.at[0])
                pltpu.sync_copy(ring.at[0], out_hbm.at[cid, pl.ds(r0, CHUNK)])

    return degree, edge_agg


def _enc_body(deg_ref, x_ref, ewt_ref, eb_ref, w0t_ref,
              dinv_ref, u0_ref, s0_ref, m0_ref):
    deg = deg_ref[:, 0:1] + deg_ref[:, 1:2] + 1.0
    dinv = lax.rsqrt(deg)
    dinv_ref[...] = dinv
    h0 = jnp.dot(x_ref[...], ewt_ref[...],
                 preferred_element_type=jnp.float32) + eb_ref[...]
    s0_ref[...] = jnp.sum(h0, axis=0, keepdims=True)
    m0_ref[...] = jnp.max(h0, axis=0, keepdims=True)
    u0_ref[...] = jnp.dot(h0, w0t_ref[...],
                          preferred_element_type=jnp.float32) * dinv


def _bn_layer(sp_ref, u_ref, dinv_ref, cb_ref, g_ref, b_ref):
    s = sp_ref[0, :N, :] + sp_ref[1, :N, :]
    agg = dinv_ref[...] * (s + u_ref[...]) + cb_ref[...]
    mean = jnp.mean(agg, axis=0, keepdims=True)
    cen = agg - mean
    var = jnp.mean(cen * cen, axis=0, keepdims=True)
    hn = cen * lax.rsqrt(var + BN_EPS) * g_ref[...] + b_ref[...]
    return jnp.maximum(hn, 0.0)


def _mid_body(sp_ref, u_ref, dinv_ref, cb_ref, g_ref, b_ref, wt_ref,
              unext_ref, s_ref, m_ref):
    h = _bn_layer(sp_ref, u_ref, dinv_ref, cb_ref, g_ref, b_ref)
    s_ref[...] = jnp.sum(h, axis=0, keepdims=True)
    m_ref[...] = jnp.max(h, axis=0, keepdims=True)
    unext_ref[...] = jnp.dot(h, wt_ref[...],
                             preferred_element_type=jnp.float32) * dinv_ref[...]


def _fin_body(sp_ref, u_ref, dinv_ref, cb_ref, g_ref, b_ref,
              ss_ref, mm_ref, wms_ref, wx_ref, b1_ref, w2t_ref, b2_ref,
              out_ref):
    h = _bn_layer(sp_ref, u_ref, dinv_ref, cb_ref, g_ref, b_ref)
    s4 = jnp.sum(h, axis=0, keepdims=True)
    m4 = jnp.max(h, axis=0, keepdims=True)
    acc = (jnp.dot(s4, wms_ref[L], preferred_element_type=jnp.float32) +
           jnp.dot(m4, wx_ref[L], preferred_element_type=jnp.float32))
    for i in range(L):
        acc += jnp.dot(ss_ref[pl.ds(i, 1), :], wms_ref[i],
                       preferred_element_type=jnp.float32)
        acc += jnp.dot(mm_ref[pl.ds(i, 1), :], wx_ref[i],
                       preferred_element_type=jnp.float32)
    h1 = jnp.maximum(acc + b1_ref[...], 0.0)
    out_ref[...] = jnp.dot(h1, w2t_ref[...],
                           preferred_element_type=jnp.float32) + b2_ref[...]


_f32 = jnp.float32
_enc_call = pl.pallas_call(
    _enc_body,
    out_shape=(
        jax.ShapeDtypeStruct((N, 1), _f32),
        jax.ShapeDtypeStruct((N, H), _f32),
        jax.ShapeDtypeStruct((1, H), _f32),
        jax.ShapeDtypeStruct((1, H), _f32),
    ),
)
_mid_call = pl.pallas_call(
    _mid_body,
    out_shape=(
        jax.ShapeDtypeStruct((N, H), _f32),
        jax.ShapeDtypeStruct((1, H), _f32),
        jax.ShapeDtypeStruct((1, H), _f32),
    ),
)
_fin_call = pl.pallas_call(
    _fin_body,
    out_shape=jax.ShapeDtypeStruct((1, 1), _f32),
)


def kernel(x, edge_index, params):
    info = plsc.get_sparse_core_info()
    nc, ns = info.num_cores, info.num_subcores
    nw, ch, ch_deg, tot_ch, tot_ch_pad, n_pad = _geom(nc, ns)
    degree_call, agg_call = _sc_calls(nc, ns)

    src = edge_index[0]
    dst = edge_index[1]
    pad = tot_ch_pad * CHUNK - E
    srcp = jnp.concatenate([src, jnp.zeros((pad,), src.dtype)])
    dstp = jnp.concatenate([dst, jnp.full((pad,), N, dst.dtype)])
    src_w = srcp.reshape(tot_ch_pad, CHUNK)
    dst_w = dstp.reshape(tot_ch_pad, CHUNK)

    deg_out = degree_call(dst_w)
    deg_cols = deg_out[:, :N, 0].T

    p = params
    enc_WT = p['enc_W'].T
    eb = p['enc_b'][None, :]
    convWT = [p['conv_W'][i].T for i in range(L)]
    cb = [p['conv_b'][i][None, :] for i in range(L)]
    bng = [p['bn_g'][i][None, :] for i in range(L)]
    bnb = [p['bn_b'][i][None, :] for i in range(L)]

    K = H * (L + 1)
    W1 = p['out1_W']
    Wm = W1[:, :K].reshape(H, L + 1, H)
    Ws = W1[:, K:2 * K].reshape(H, L + 1, H)
    Wx = W1[:, 2 * K:].reshape(H, L + 1, H)
    Wms_T = jnp.transpose(Wm * (1.0 / N) + Ws, (1, 2, 0))
    Wx_T = jnp.transpose(Wx, (1, 2, 0))
    b1 = p['out1_b'][None, :]
    w2t = p['out2_W'].T
    b2 = p['out2_b'][None, :]

    dinv, u, s0, m0 = _enc_call(deg_cols, x, enc_WT, eb, convWT[0])
    sums, maxs = [s0], [m0]
    for i in range(L - 1):
        S = agg_call(src_w, dst_w, u)
        u, si, mi = _mid_call(S, u, dinv, cb[i], bng[i], bnb[i], convWT[i + 1])
        sums.append(si)
        maxs.append(mi)
    S = agg_call(src_w, dst_w, u)
    ss = jnp.concatenate(sums, axis=0)
    mm = jnp.concatenate(maxs, axis=0)
    return _fin_call(S, u, dinv, cb[L - 1], bng[L - 1], bnb[L - 1],
                     ss, mm, Wms_T, Wx_T, b1, w2t, b2)


# speedup vs baseline: 2.2401x; 2.2401x over previous
import functools

import jax
import jax.numpy as jnp
from jax import lax
from jax.experimental import pallas as pl
from jax.experimental.pallas import tpu as pltpu
from jax.experimental.pallas import tpu_sc as plsc

N = 10000
E = 320000
D_IN = 128
H = 64
L = 4
CHUNK = 128
NB = 4
DEG_W = 16
BN_EPS = 1e-5
CORE0_SHARE = 0.5


def _geom(nc, ns):
    nw = nc * ns
    n_chunks = -(-E // CHUNK)
    ch = [0] * nc
    ch[0] = max(NB, round(n_chunks * CORE0_SHARE / ns / NB) * NB)
    if nc > 1:
        rest = max(0, n_chunks - ns * ch[0])
        per = -(-rest // ((nc - 1) * ns))
        cho = max(NB, -(-per // NB) * NB)
        for c in range(1, nc):
            ch[c] = cho
    tot_ch = ns * sum(ch)
    tot_ch_pad = tot_ch + max(ch)
    ch_deg = -(-tot_ch // nw)
    tot_ch_pad = max(tot_ch_pad, nw * ch_deg)
    n_pad = ns * CHUNK * (-(-(N + 1) // (ns * CHUNK)))
    return nw, tuple(ch), ch_deg, tot_ch, tot_ch_pad, n_pad


@functools.cache
def _sc_calls(nc, ns):
    nw, ch, ch_deg, tot_ch, tot_ch_pad, n_pad = _geom(nc, ns)
    ch_max = max(ch)
    rpt = n_pad // ns
    nzc = rpt // CHUNK
    mesh = plsc.VectorSubcoreMesh(
        core_axis_name="c", subcore_axis_name="s",
        num_cores=nc, num_subcores=ns)
    cparams = pltpu.CompilerParams(use_tc_tiling_on_sc=False)

    @functools.partial(
        pl.kernel,
        out_type=jax.ShapeDtypeStruct((nc, n_pad, DEG_W), jnp.float32),
        mesh=mesh,
        compiler_params=cparams,
        scratch_types=[
            pltpu.VMEM((ch_deg, CHUNK), jnp.int32),
            pltpu.VMEM((CHUNK, DEG_W), jnp.float32),
            pltpu.VMEM_SHARED((n_pad, DEG_W), jnp.float32),
        ],
    )
    def degree(dst_hbm, out_hbm, dst_v, val_v, acc):
        cid = lax.axis_index("c")
        sid = lax.axis_index("s")
        wid = sid * nc + cid
        pltpu.sync_copy(dst_hbm.at[pl.ds(wid * ch_deg, ch_deg)], dst_v)

        def _fill(c):
            v = jnp.full((16,), c, jnp.float32)

            def _f(i, _):
                val_v[i, pl.ds(0, 16)] = v
                return 0

            lax.fori_loop(0, CHUNK, _f, 0)

        _fill(0.0)
        for q in range(nzc):
            pltpu.sync_copy(val_v, acc.at[pl.ds(sid * rpt + q * CHUNK, CHUNK)])
        plsc.subcore_barrier()

        _fill(1.0)

        def _step(j, _):
            pltpu.sync_copy(val_v, acc.at[dst_v.at[j]], add=True)
            return 0

        lax.fori_loop(0, ch_deg, _step, 0)
        plsc.subcore_barrier()

        for q in range(nzc):
            r0 = sid * rpt + q * CHUNK
            pltpu.sync_copy(acc.at[pl.ds(r0, CHUNK)], val_v)
            pltpu.sync_copy(val_v, out_hbm.at[cid, pl.ds(r0, CHUNK)])

    @functools.partial(
        pl.kernel,
        out_type=jax.ShapeDtypeStruct((nc, n_pad, H), jnp.float32),
        mesh=mesh,
        compiler_params=cparams,
        scratch_types=[
            pltpu.VMEM((ch_max, CHUNK), jnp.int32),
            pltpu.VMEM((ch_max, CHUNK), jnp.int32),
            pltpu.VMEM((NB, CHUNK, H), jnp.float32),
            pltpu.VMEM_SHARED((n_pad, H), jnp.float32),
            pltpu.SemaphoreType.DMA,
        ],
    )
    def edge_agg(src_hbm, dst_hbm, u_hbm, out_hbm, src_v, dst_v, ring, acc, sem):
        cid = lax.axis_index("c")
        sid = lax.axis_index("s")
        core_base = 0
        ch_w = jnp.int32(ch[0])
        base_ch = sid * ch[0]
        for c in range(1, nc):
            core_base += ns * ch[c - 1]
            ch_w = jnp.where(cid == c, jnp.int32(ch[c]), ch_w)
            base_ch = jnp.where(cid == c, core_base + sid * ch[c], base_ch)
        with jax.named_scope("stage_idx"):
            for c in range(nc):
                if ch[c] > 0:
                    @pl.when(cid == c)
                    def _():
                        pltpu.sync_copy(src_hbm.at[pl.ds(base_ch, ch[c])],
                                        src_v.at[pl.ds(0, ch[c])])
                        pltpu.sync_copy(dst_hbm.at[pl.ds(base_ch, ch[c])],
                                        dst_v.at[pl.ds(0, ch[c])])

        with jax.named_scope("zero_acc"):
            zero = jnp.zeros((16,), jnp.float32)

            def _zf(t, _):
                i = t // (H // 16)
                k = t % (H // 16)
                ring[0, i, pl.ds(k * 16, 16)] = zero
                return 0

            lax.fori_loop(0, CHUNK * (H // 16), _zf, 0)
            for q in range(nzc):
                pltpu.sync_copy(ring.at[0],
                                acc.at[pl.ds(sid * rpt + q * CHUNK, CHUNK)])
            plsc.subcore_barrier()

        with jax.named_scope("edges"):
            for b in range(NB):
                pltpu.async_copy(u_hbm.at[src_v.at[b]], ring.at[b], sem)

            def _step(g, _):
                for b in range(NB):
                    j = g * NB + b
                    pltpu.make_async_copy(
                        u_hbm.at[pl.ds(0, CHUNK)], ring.at[b], sem).wait()
                    pltpu.sync_copy(ring.at[b], acc.at[dst_v.at[j]], add=True)

                    @pl.when(j + NB < ch_w)
                    def _():
                        pltpu.async_copy(
                            u_hbm.at[src_v.at[j + NB]], ring.at[b], sem)

                return 0

            lax.fori_loop(0, ch_w // NB, _step, 0)
            plsc.subcore_barrier()

        with jax.named_scope("readout"):
            for q in range(nzc):
                r0 = sid * rpt + q * CHUNK
                pltpu.sync_copy(acc.at[pl.ds(r0, CHUNK)], ring.at[0])
                pltpu.sync_copy(ring.at[0], out_hbm.at[cid, pl.ds(r0, CHUNK)])

    return degree, edge_agg


def _enc_body(deg_ref, x_ref, ewt_ref, eb_ref, w0t_ref,
              dinv_ref, u0_ref, s0_ref, m0_ref):
    deg = deg_ref[:, 0:1] + deg_ref[:, 1:2] + 1.0
    dinv = lax.rsqrt(deg)
    dinv_ref[...] = dinv
    h0 = jnp.dot(x_ref[...], ewt_ref[...],
                 preferred_element_type=jnp.float32) + eb_ref[...]
    s0_ref[...] = jnp.sum(h0, axis=0, keepdims=True)
    m0_ref[...] = jnp.max(h0, axis=0, keepdims=True)
    u0_ref[...] = jnp.dot(h0, w0t_ref[...],
                          preferred_element_type=jnp.float32) * dinv


def _bn_layer(sp_ref, u_ref, dinv_ref, cb_ref, g_ref, b_ref):
    s = sp_ref[0, :N, :] + sp_ref[1, :N, :]
    agg = dinv_ref[...] * (s + u_ref[...]) + cb_ref[...]
    mean = jnp.mean(agg, axis=0, keepdims=True)
    cen = agg - mean
    var = jnp.mean(cen * cen, axis=0, keepdims=True)
    hn = cen * lax.rsqrt(var + BN_EPS) * g_ref[...] + b_ref[...]
    return jnp.maximum(hn, 0.0)


def _mid_body(sp_ref, u_ref, dinv_ref, cb_ref, g_ref, b_ref, wt_ref,
              unext_ref, s_ref, m_ref):
    h = _bn_layer(sp_ref, u_ref, dinv_ref, cb_ref, g_ref, b_ref)
    s_ref[...] = jnp.sum(h, axis=0, keepdims=True)
    m_ref[...] = jnp.max(h, axis=0, keepdims=True)
    unext_ref[...] = jnp.dot(h, wt_ref[...],
                             preferred_element_type=jnp.float32) * dinv_ref[...]


def _fin_body(sp_ref, u_ref, dinv_ref, cb_ref, g_ref, b_ref,
              ss_ref, mm_ref, wms_ref, wx_ref, b1_ref, w2t_ref, b2_ref,
              out_ref):
    h = _bn_layer(sp_ref, u_ref, dinv_ref, cb_ref, g_ref, b_ref)
    s4 = jnp.sum(h, axis=0, keepdims=True)
    m4 = jnp.max(h, axis=0, keepdims=True)
    acc = (jnp.dot(s4, wms_ref[L], preferred_element_type=jnp.float32) +
           jnp.dot(m4, wx_ref[L], preferred_element_type=jnp.float32))
    for i in range(L):
        acc += jnp.dot(ss_ref[pl.ds(i, 1), :], wms_ref[i],
                       preferred_element_type=jnp.float32)
        acc += jnp.dot(mm_ref[pl.ds(i, 1), :], wx_ref[i],
                       preferred_element_type=jnp.float32)
    h1 = jnp.maximum(acc + b1_ref[...], 0.0)
    out_ref[...] = jnp.dot(h1, w2t_ref[...],
                           preferred_element_type=jnp.float32) + b2_ref[...]


_f32 = jnp.float32
_enc_call = pl.pallas_call(
    _enc_body,
    out_shape=(
        jax.ShapeDtypeStruct((N, 1), _f32),
        jax.ShapeDtypeStruct((N, H), _f32),
        jax.ShapeDtypeStruct((1, H), _f32),
        jax.ShapeDtypeStruct((1, H), _f32),
    ),
)
_mid_call = pl.pallas_call(
    _mid_body,
    out_shape=(
        jax.ShapeDtypeStruct((N, H), _f32),
        jax.ShapeDtypeStruct((1, H), _f32),
        jax.ShapeDtypeStruct((1, H), _f32),
    ),
)
_fin_call = pl.pallas_call(
    _fin_body,
    out_shape=jax.ShapeDtypeStruct((1, 1), _f32),
)


def kernel(x, edge_index, params):
    info = plsc.get_sparse_core_info()
    nc, ns = info.num_cores, info.num_subcores
    nw, ch, ch_deg, tot_ch, tot_ch_pad, n_pad = _geom(nc, ns)
    degree_call, agg_call = _sc_calls(nc, ns)

    src = edge_index[0]
    dst = edge_index[1]
    pad = tot_ch_pad * CHUNK - E
    ar = jnp.arange(pad, dtype=src.dtype)
    srcp = jnp.concatenate([src, ar % N])
    dstp = jnp.concatenate([dst, N + ar % (n_pad - N)])
    src_w = srcp.reshape(tot_ch_pad, CHUNK)
    dst_w = dstp.reshape(tot_ch_pad, CHUNK)

    deg_out = degree_call(dst_w)
    deg_cols = deg_out[:, :N, 0].T

    p = params
    enc_WT = p['enc_W'].T
    eb = p['enc_b'][None, :]
    convWT = [p['conv_W'][i].T for i in range(L)]
    cb = [p['conv_b'][i][None, :] for i in range(L)]
    bng = [p['bn_g'][i][None, :] for i in range(L)]
    bnb = [p['bn_b'][i][None, :] for i in range(L)]

    K = H * (L + 1)
    W1 = p['out1_W']
    Wm = W1[:, :K].reshape(H, L + 1, H)
    Ws = W1[:, K:2 * K].reshape(H, L + 1, H)
    Wx = W1[:, 2 * K:].reshape(H, L + 1, H)
    Wms_T = jnp.transpose(Wm * (1.0 / N) + Ws, (1, 2, 0))
    Wx_T = jnp.transpose(Wx, (1, 2, 0))
    b1 = p['out1_b'][None, :]
    w2t = p['out2_W'].T
    b2 = p['out2_b'][None, :]

    dinv, u, s0, m0 = _enc_call(deg_cols, x, enc_WT, eb, convWT[0])
    sums, maxs = [s0], [m0]
    for i in range(L - 1):
        S = agg_call(src_w, dst_w, u)
        u, si, mi = _mid_call(S, u, dinv, cb[i], bng[i], bnb[i], convWT[i + 1])
        sums.append(si)
        maxs.append(mi)
    S = agg_call(src_w, dst_w, u)
    ss = jnp.concatenate(sums, axis=0)
    mm = jnp.concatenate(maxs, axis=0)
    return _fin_call(S, u, dinv, cb[L - 1], bng[L - 1], bnb[L - 1],
                     ss, mm, Wms_T, Wx_T, b1, w2t, b2)


# speedup vs baseline: 2.5966x; 1.1592x over previous
import functools

import jax
import jax.numpy as jnp
from jax import lax
from jax.experimental import pallas as pl
from jax.experimental.pallas import tpu as pltpu
from jax.experimental.pallas import tpu_sc as plsc

N = 10000
E = 320000
D_IN = 128
H = 64
L = 4
CHUNK = 128
NB = 8
DEG_W = 16
BN_EPS = 1e-5
CORE0_SHARE = 0.5


def _geom(nc, ns):
    nw = nc * ns
    n_chunks = -(-E // CHUNK)
    ch = [0] * nc
    ch[0] = max(NB, round(n_chunks * CORE0_SHARE / ns / NB) * NB)
    if nc > 1:
        rest = max(0, n_chunks - ns * ch[0])
        per = -(-rest // ((nc - 1) * ns))
        cho = max(NB, -(-per // NB) * NB)
        for c in range(1, nc):
            ch[c] = cho
    tot_ch = ns * sum(ch)
    tot_ch_pad = tot_ch + max(ch)
    ch_deg = -(-tot_ch // nw)
    tot_ch_pad = max(tot_ch_pad, nw * ch_deg)
    n_pad = ns * CHUNK * (-(-(N + 1) // (ns * CHUNK)))
    return nw, tuple(ch), ch_deg, tot_ch, tot_ch_pad, n_pad


@functools.cache
def _sc_calls(nc, ns):
    nw, ch, ch_deg, tot_ch, tot_ch_pad, n_pad = _geom(nc, ns)
    ch_max = max(ch)
    rpt = n_pad // ns
    nzc = rpt // CHUNK
    mesh = plsc.VectorSubcoreMesh(
        core_axis_name="c", subcore_axis_name="s",
        num_cores=nc, num_subcores=ns)
    cparams = pltpu.CompilerParams(use_tc_tiling_on_sc=False)

    @functools.partial(
        pl.kernel,
        out_type=jax.ShapeDtypeStruct((nc, n_pad, DEG_W), jnp.float32),
        mesh=mesh,
        compiler_params=cparams,
        scratch_types=[
            pltpu.VMEM((ch_deg, CHUNK), jnp.int32),
            pltpu.VMEM((CHUNK, DEG_W), jnp.float32),
            pltpu.VMEM_SHARED((n_pad, DEG_W), jnp.float32),
        ],
    )
    def degree(dst_hbm, out_hbm, dst_v, val_v, acc):
        cid = lax.axis_index("c")
        sid = lax.axis_index("s")
        wid = sid * nc + cid
        pltpu.sync_copy(dst_hbm.at[pl.ds(wid * ch_deg, ch_deg)], dst_v)

        def _fill(c):
            v = jnp.full((16,), c, jnp.float32)

            def _f(i, _):
                val_v[i, pl.ds(0, 16)] = v
                return 0

            lax.fori_loop(0, CHUNK, _f, 0)

        _fill(0.0)
        for q in range(nzc):
            pltpu.sync_copy(val_v, acc.at[pl.ds(sid * rpt + q * CHUNK, CHUNK)])
        plsc.subcore_barrier()

        _fill(1.0)

        def _step(j, _):
            pltpu.sync_copy(val_v, acc.at[dst_v.at[j]], add=True)
            return 0

        lax.fori_loop(0, ch_deg, _step, 0)
        plsc.subcore_barrier()

        for q in range(nzc):
            r0 = sid * rpt + q * CHUNK
            pltpu.sync_copy(acc.at[pl.ds(r0, CHUNK)], val_v)
            pltpu.sync_copy(val_v, out_hbm.at[cid, pl.ds(r0, CHUNK)])

    @functools.partial(
        pl.kernel,
        out_type=jax.ShapeDtypeStruct((nc, n_pad, H), jnp.float32),
        mesh=mesh,
        compiler_params=cparams,
        scratch_types=[
            pltpu.VMEM((ch_max, CHUNK), jnp.int32),
            pltpu.VMEM((ch_max, CHUNK), jnp.int32),
            pltpu.VMEM((NB, CHUNK, H), jnp.float32),
            pltpu.VMEM_SHARED((n_pad, H), jnp.float32),
            pltpu.SemaphoreType.DMA,
        ],
    )
    def edge_agg(src_hbm, dst_hbm, u_hbm, out_hbm, src_v, dst_v, ring, acc, sem):
        cid = lax.axis_index("c")
        sid = lax.axis_index("s")
        core_base = 0
        ch_w = jnp.int32(ch[0])
        base_ch = sid * ch[0]
        for c in range(1, nc):
            core_base += ns * ch[c - 1]
            ch_w = jnp.where(cid == c, jnp.int32(ch[c]), ch_w)
            base_ch = jnp.where(cid == c, core_base + sid * ch[c], base_ch)
        with jax.named_scope("stage_idx"):
            for c in range(nc):
                if ch[c] > 0:
                    @pl.when(cid == c)
                    def _():
                        pltpu.sync_copy(src_hbm.at[pl.ds(base_ch, ch[c])],
                                        src_v.at[pl.ds(0, ch[c])])
                        pltpu.sync_copy(dst_hbm.at[pl.ds(base_ch, ch[c])],
                                        dst_v.at[pl.ds(0, ch[c])])

        with jax.named_scope("zero_acc"):
            zero = jnp.zeros((16,), jnp.float32)

            def _zf(t, _):
                i = t // (H // 16)
                k = t % (H // 16)
                ring[0, i, pl.ds(k * 16, 16)] = zero
                return 0

            lax.fori_loop(0, CHUNK * (H // 16), _zf, 0)
            for q in range(nzc):
                pltpu.sync_copy(ring.at[0],
                                acc.at[pl.ds(sid * rpt + q * CHUNK, CHUNK)])
            plsc.subcore_barrier()

        with jax.named_scope("edges"):
            for b in range(NB):
                pltpu.async_copy(u_hbm.at[src_v.at[b]], ring.at[b], sem)

            def _step(g, _):
                for b in range(NB):
                    j = g * NB + b
                    pltpu.make_async_copy(
                        u_hbm.at[pl.ds(0, CHUNK)], ring.at[b], sem).wait()
                    pltpu.sync_copy(ring.at[b], acc.at[dst_v.at[j]], add=True)

                    @pl.when(j + NB < ch_w)
                    def _():
                        pltpu.async_copy(
                            u_hbm.at[src_v.at[j + NB]], ring.at[b], sem)

                return 0

            lax.fori_loop(0, ch_w // NB, _step, 0)
            plsc.subcore_barrier()

        with jax.named_scope("readout"):
            for q in range(nzc):
                r0 = sid * rpt + q * CHUNK
                pltpu.sync_copy(acc.at[pl.ds(r0, CHUNK)], ring.at[0])
                pltpu.sync_copy(ring.at[0], out_hbm.at[cid, pl.ds(r0, CHUNK)])

    return degree, edge_agg


def _dot_t(a, b):
    return lax.dot_general(a, b, (((1,), (1,)), ((), ())),
                           preferred_element_type=jnp.float32)


def _enc_body(deg_ref, x_ref, ew_ref, eb_ref, w0_ref,
              dinv_ref, u0_ref, s0_ref, m0_ref):
    degc = lax.dot_general(deg_ref[...], jnp.ones((2, 1), jnp.float32),
                           (((0,), (0,)), ((), ())),
                           preferred_element_type=jnp.float32)
    deg = degc[:N, :] + 1.0
    dinv = lax.rsqrt(deg)
    dinv_ref[...] = dinv
    h0 = _dot_t(x_ref[...], ew_ref[...]) + eb_ref[...]
    s0_ref[...] = jnp.sum(h0, axis=0, keepdims=True)
    m0_ref[...] = jnp.max(h0, axis=0, keepdims=True)
    u0_ref[...] = _dot_t(h0, w0_ref[...]) * dinv


def _bn_layer(sp_ref, u_ref, dinv_ref, cb_ref, g_ref, b_ref):
    s = sp_ref[0, :N, :] + sp_ref[1, :N, :]
    agg = dinv_ref[...] * (s + u_ref[...]) + cb_ref[...]
    mean = jnp.mean(agg, axis=0, keepdims=True)
    cen = agg - mean
    var = jnp.mean(cen * cen, axis=0, keepdims=True)
    hn = cen * lax.rsqrt(var + BN_EPS) * g_ref[...] + b_ref[...]
    return jnp.maximum(hn, 0.0)


def _mid_body(sp_ref, u_ref, dinv_ref, cb_ref, g_ref, b_ref, w_ref,
              unext_ref, s_ref, m_ref):
    h = _bn_layer(sp_ref, u_ref, dinv_ref, cb_ref, g_ref, b_ref)
    s_ref[...] = jnp.sum(h, axis=0, keepdims=True)
    m_ref[...] = jnp.max(h, axis=0, keepdims=True)
    unext_ref[...] = _dot_t(h, w_ref[...]) * dinv_ref[...]


def _fin_body(sp_ref, u_ref, dinv_ref, cb_ref, g_ref, b_ref,
              ss_ref, mm_ref, *rest):
    wm = rest[:L + 1]
    ws = rest[L + 1:2 * (L + 1)]
    wx = rest[2 * (L + 1):3 * (L + 1)]
    b1_ref, w2_ref, b2_ref, out_ref = rest[3 * (L + 1):]
    h = _bn_layer(sp_ref, u_ref, dinv_ref, cb_ref, g_ref, b_ref)
    s4 = jnp.sum(h, axis=0, keepdims=True)
    m4 = jnp.max(h, axis=0, keepdims=True)
    acc = b1_ref[...]
    for i in range(L + 1):
        s_i = s4 if i == L else ss_ref[pl.ds(i, 1), :]
        m_i = m4 if i == L else mm_ref[pl.ds(i, 1), :]
        acc = acc + _dot_t(s_i * (1.0 / N), wm[i][...])
        acc = acc + _dot_t(s_i, ws[i][...])
        acc = acc + _dot_t(m_i, wx[i][...])
    h1 = jnp.maximum(acc, 0.0)
    out_ref[...] = (jnp.sum(h1 * w2_ref[...], axis=1, keepdims=True)
                    + b2_ref[...])


_f32 = jnp.float32
_enc_call = pl.pallas_call(
    _enc_body,
    out_shape=(
        jax.ShapeDtypeStruct((N, 1), _f32),
        jax.ShapeDtypeStruct((N, H), _f32),
        jax.ShapeDtypeStruct((1, H), _f32),
        jax.ShapeDtypeStruct((1, H), _f32),
    ),
)
_mid_call = pl.pallas_call(
    _mid_body,
    out_shape=(
        jax.ShapeDtypeStruct((N, H), _f32),
        jax.ShapeDtypeStruct((1, H), _f32),
        jax.ShapeDtypeStruct((1, H), _f32),
    ),
)
_fin_call = pl.pallas_call(
    _fin_body,
    out_shape=jax.ShapeDtypeStruct((1, 1), _f32),
)


def kernel(x, edge_index, params):
    info = plsc.get_sparse_core_info()
    nc, ns = info.num_cores, info.num_subcores
    nw, ch, ch_deg, tot_ch, tot_ch_pad, n_pad = _geom(nc, ns)
    degree_call, agg_call = _sc_calls(nc, ns)

    src = edge_index[0]
    dst = edge_index[1]
    pad = tot_ch_pad * CHUNK - E
    ar = jnp.arange(pad, dtype=src.dtype)
    srcp = jnp.concatenate([src, ar % N])
    dstp = jnp.concatenate([dst, N + ar % (n_pad - N)])
    src_w = srcp.reshape(tot_ch_pad, CHUNK)
    dst_w = dstp.reshape(tot_ch_pad, CHUNK)

    deg_out = degree_call(dst_w)
    deg2 = deg_out[:, :, 0]

    p = params
    eb = p['enc_b'][None, :]
    convW = [p['conv_W'][i] for i in range(L)]
    cb = [p['conv_b'][i][None, :] for i in range(L)]
    bng = [p['bn_g'][i][None, :] for i in range(L)]
    bnb = [p['bn_b'][i][None, :] for i in range(L)]
    b1 = p['out1_b'][None, :]
    b2 = p['out2_b'][None, :]

    dinv, u, s0, m0 = _enc_call(deg2, x, p['enc_W'], eb, convW[0])
    sums, maxs = [s0], [m0]
    for i in range(L - 1):
        S = agg_call(src_w, dst_w, u)
        u, si, mi = _mid_call(S, u, dinv, cb[i], bng[i], bnb[i], convW[i + 1])
        sums.append(si)
        maxs.append(mi)
    S = agg_call(src_w, dst_w, u)
    ss = jnp.concatenate(sums, axis=0)
    mm = jnp.concatenate(maxs, axis=0)
    K = H * (L + 1)
    W1 = p['out1_W']
    wblocks = ([W1[:, i * H:(i + 1) * H] for i in range(L + 1)] +
               [W1[:, K + i * H:K + (i + 1) * H] for i in range(L + 1)] +
               [W1[:, 2 * K + i * H:2 * K + (i + 1) * H] for i in range(L + 1)])
    return _fin_call(S, u, dinv, cb[L - 1], bng[L - 1], bnb[L - 1],
                     ss, mm, *wblocks, b1, p['out2_W'], b2)


# speedup vs baseline: 3.3492x; 1.2898x over previous
import functools

import jax
import jax.numpy as jnp
from jax import lax
from jax.experimental import pallas as pl
from jax.experimental.pallas import tpu as pltpu
from jax.experimental.pallas import tpu_sc as plsc

N = 10000
E = 320000
D_IN = 128
H = 64
L = 4
CHUNK = 128
NB = 8
DEG_W = 16
BN_EPS = 1e-5
CORE0_SHARE = 0.5


def _geom(nc, ns):
    nw = nc * ns
    n_chunks = -(-E // CHUNK)
    ch = [0] * nc
    ch[0] = max(NB, round(n_chunks * CORE0_SHARE / ns / NB) * NB)
    if nc > 1:
        rest = max(0, n_chunks - ns * ch[0])
        per = -(-rest // ((nc - 1) * ns))
        cho = max(NB, -(-per // NB) * NB)
        for c in range(1, nc):
            ch[c] = cho
    tot_ch = ns * sum(ch)
    tot_ch_pad = tot_ch + max(ch)
    ch_deg = -(-tot_ch // nw)
    tot_ch_pad = max(tot_ch_pad, nw * ch_deg)
    n_pad = ns * CHUNK * (-(-(N + 1) // (ns * CHUNK)))
    return nw, tuple(ch), ch_deg, tot_ch, tot_ch_pad, n_pad


@functools.cache
def _sc_calls(nc, ns):
    nw, ch, ch_deg, tot_ch, tot_ch_pad, n_pad = _geom(nc, ns)
    ch_max = max(ch)
    rpt = n_pad // ns
    nzc = rpt // CHUNK
    mesh = plsc.VectorSubcoreMesh(
        core_axis_name="c", subcore_axis_name="s",
        num_cores=nc, num_subcores=ns)
    cparams = pltpu.CompilerParams(use_tc_tiling_on_sc=False)
    cparams_deg = pltpu.CompilerParams(use_tc_tiling_on_sc=False,
                                       needs_layout_passes=False)

    @functools.partial(
        pl.kernel,
        out_type=jax.ShapeDtypeStruct((nc, n_pad // 2, 2 * H), jnp.float32),
        mesh=mesh,
        compiler_params=cparams_deg,
        scratch_types=[
            pltpu.VMEM((ch_deg, CHUNK), jnp.int32),
            pltpu.VMEM((CHUNK, DEG_W), jnp.float32),
            pltpu.VMEM((rpt, DEG_W), jnp.float32),
            pltpu.VMEM((rpt // 2, 2 * H), jnp.float32),
            pltpu.VMEM_SHARED((n_pad, DEG_W), jnp.float32),
        ],
    )
    def degree(dst_hbm, out_hbm, dst_v, val_v, bnc_v, pk_v, acc):
        cid = lax.axis_index("c")
        sid = lax.axis_index("s")
        wid = sid * nc + cid
        pltpu.sync_copy(dst_hbm.at[pl.ds(wid * ch_deg, ch_deg)], dst_v)

        def _fill(c):
            v = jnp.full((16,), c, jnp.float32)

            def _f(i, _):
                val_v[i, pl.ds(0, 16)] = v
                return 0

            lax.fori_loop(0, CHUNK, _f, 0)

        _fill(0.0)
        for q in range(nzc):
            pltpu.sync_copy(val_v, acc.at[pl.ds(sid * rpt + q * CHUNK, CHUNK)])
        plsc.subcore_barrier()

        _fill(1.0)

        def _step(j, _):
            pltpu.sync_copy(val_v, acc.at[dst_v.at[j]], add=True)
            return 0

        lax.fori_loop(0, ch_deg, _step, 0)
        plsc.subcore_barrier()

        pltpu.sync_copy(acc.at[pl.ds(sid * rpt, rpt)], bnc_v)

        zi = jnp.zeros((16,), jnp.int32)

        def _pk(i, _):
            v = plsc.load_gather(bnc_v, [zi + i, zi])
            r = i // 2
            c0 = (i % 2) * H
            for k in range(H // 16):
                pk_v[r, pl.ds(c0 + k * 16, 16)] = v
            return 0

        lax.fori_loop(0, rpt, _pk, 0)
        pltpu.sync_copy(pk_v, out_hbm.at[cid, pl.ds(sid * (rpt // 2), rpt // 2)])

    @functools.partial(
        pl.kernel,
        out_type=jax.ShapeDtypeStruct((nc, n_pad, H), jnp.float32),
        mesh=mesh,
        compiler_params=cparams,
        scratch_types=[
            pltpu.VMEM((ch_max, CHUNK), jnp.int32),
            pltpu.VMEM((ch_max, CHUNK), jnp.int32),
            pltpu.VMEM((NB, CHUNK, H), jnp.float32),
            pltpu.VMEM_SHARED((n_pad, H), jnp.float32),
            pltpu.SemaphoreType.DMA,
        ],
    )
    def edge_agg(src_hbm, dst_hbm, u_hbm, out_hbm, src_v, dst_v, ring, acc, sem):
        cid = lax.axis_index("c")
        sid = lax.axis_index("s")
        core_base = 0
        ch_w = jnp.int32(ch[0])
        base_ch = sid * ch[0]
        for c in range(1, nc):
            core_base += ns * ch[c - 1]
            ch_w = jnp.where(cid == c, jnp.int32(ch[c]), ch_w)
            base_ch = jnp.where(cid == c, core_base + sid * ch[c], base_ch)
        with jax.named_scope("stage_idx"):
            for c in range(nc):
                if ch[c] > 0:
                    @pl.when(cid == c)
                    def _():
                        pltpu.sync_copy(src_hbm.at[pl.ds(base_ch, ch[c])],
                                        src_v.at[pl.ds(0, ch[c])])
                        pltpu.sync_copy(dst_hbm.at[pl.ds(base_ch, ch[c])],
                                        dst_v.at[pl.ds(0, ch[c])])

        with jax.named_scope("zero_acc"):
            zero = jnp.zeros((16,), jnp.float32)

            def _zf(t, _):
                i = t // (H // 16)
                k = t % (H // 16)
                ring[0, i, pl.ds(k * 16, 16)] = zero
                return 0

            lax.fori_loop(0, CHUNK * (H // 16), _zf, 0)
            for q in range(nzc):
                pltpu.sync_copy(ring.at[0],
                                acc.at[pl.ds(sid * rpt + q * CHUNK, CHUNK)])
            plsc.subcore_barrier()

        with jax.named_scope("edges"):
            for b in range(NB):
                pltpu.async_copy(u_hbm.at[src_v.at[b]], ring.at[b], sem)

            def _step(g, _):
                for b in range(NB):
                    j = g * NB + b
                    pltpu.make_async_copy(
                        u_hbm.at[pl.ds(0, CHUNK)], ring.at[b], sem).wait()
                    pltpu.sync_copy(ring.at[b], acc.at[dst_v.at[j]], add=True)

                    @pl.when(j + NB < ch_w)
                    def _():
                        pltpu.async_copy(
                            u_hbm.at[src_v.at[j + NB]], ring.at[b], sem)

                return 0

            lax.fori_loop(0, ch_w // NB, _step, 0)
            plsc.subcore_barrier()

        with jax.named_scope("readout"):
            for q in range(nzc):
                r0 = sid * rpt + q * CHUNK
                pltpu.sync_copy(acc.at[pl.ds(r0, CHUNK)], ring.at[0])
                pltpu.sync_copy(ring.at[0], out_hbm.at[cid, pl.ds(r0, CHUNK)])

    return degree, edge_agg


NP = N // 2


def _dot_t(a, b):
    return lax.dot_general(a, b, (((1,), (1,)), ((), ())),
                           preferred_element_type=jnp.float32)


def _pair(v, p_ref):
    return v + jnp.dot(v, p_ref[...], preferred_element_type=jnp.float32)


def _pairmax(v, p_ref):
    return jnp.maximum(v, jnp.dot(v, p_ref[...],
                                  preferred_element_type=jnp.float32))


def _enc_body(deg_ref, x_ref, mw_ref, eb_ref, m0_ref_w, p_ref,
              dinv_ref, u0_ref, s0_ref, m0_ref):
    deg = deg_ref[0, :NP, :] + deg_ref[1, :NP, :] + 1.0
    dinv = lax.rsqrt(deg)
    dinv_ref[...] = dinv
    h0 = _dot_t(x_ref[...], mw_ref[...]) + eb_ref[...]
    s0_ref[...] = _pair(jnp.sum(h0, axis=0, keepdims=True), p_ref)
    m0_ref[...] = _pairmax(jnp.max(h0, axis=0, keepdims=True), p_ref)
    u0_ref[...] = _dot_t(h0, m0_ref_w[...]) * dinv


def _bn_layer(sp_ref, u_ref, dinv_ref, cb_ref, g_ref, b_ref, p_ref):
    s = sp_ref[0, :NP, :] + sp_ref[1, :NP, :]
    agg = dinv_ref[...] * (s + u_ref[...]) + cb_ref[...]
    mean = _pair(jnp.sum(agg, axis=0, keepdims=True), p_ref) * (1.0 / N)
    cen = agg - mean
    var = _pair(jnp.sum(cen * cen, axis=0, keepdims=True), p_ref) * (1.0 / N)
    hn = cen * lax.rsqrt(var + BN_EPS) * g_ref[...] + b_ref[...]
    return jnp.maximum(hn, 0.0)


def _mid_body(sp_ref, u_ref, dinv_ref, cb_ref, g_ref, b_ref, w_ref, p_ref,
              unext_ref, s_ref, m_ref):
    h = _bn_layer(sp_ref, u_ref, dinv_ref, cb_ref, g_ref, b_ref, p_ref)
    s_ref[...] = _pair(jnp.sum(h, axis=0, keepdims=True), p_ref)
    m_ref[...] = _pairmax(jnp.max(h, axis=0, keepdims=True), p_ref)
    unext_ref[...] = _dot_t(h, w_ref[...]) * dinv_ref[...]


def _fin_body(sp_ref, u_ref, dinv_ref, cb_ref, g_ref, b_ref,
              ss_ref, mm_ref, p_ref, *rest):
    wm = rest[:L + 1]
    ws = rest[L + 1:2 * (L + 1)]
    wx = rest[2 * (L + 1):3 * (L + 1)]
    b1_ref, w2_ref, b2_ref, out_ref = rest[3 * (L + 1):]
    h = _bn_layer(sp_ref, u_ref, dinv_ref, cb_ref, g_ref, b_ref, p_ref)
    s4 = _pair(jnp.sum(h, axis=0, keepdims=True), p_ref)
    m4 = _pairmax(jnp.max(h, axis=0, keepdims=True), p_ref)
    acc = b1_ref[...]
    for i in range(L + 1):
        s_i = s4 if i == L else ss_ref[pl.ds(i, 1), :]
        m_i = m4 if i == L else mm_ref[pl.ds(i, 1), :]
        acc = acc + _dot_t(s_i * (1.0 / N), wm[i][...])
        acc = acc + _dot_t(s_i, ws[i][...])
        acc = acc + _dot_t(m_i, wx[i][...])
    h1 = jnp.maximum(acc, 0.0)
    out_ref[...] = (jnp.sum(h1 * w2_ref[...], axis=1, keepdims=True)
                    + b2_ref[...])


_f32 = jnp.float32
_W2 = 2 * H
_enc_call = pl.pallas_call(
    _enc_body,
    out_shape=(
        jax.ShapeDtypeStruct((NP, _W2), _f32),
        jax.ShapeDtypeStruct((NP, _W2), _f32),
        jax.ShapeDtypeStruct((1, _W2), _f32),
        jax.ShapeDtypeStruct((1, _W2), _f32),
    ),
)
_mid_call = pl.pallas_call(
    _mid_body,
    out_shape=(
        jax.ShapeDtypeStruct((NP, _W2), _f32),
        jax.ShapeDtypeStruct((1, _W2), _f32),
        jax.ShapeDtypeStruct((1, _W2), _f32),
    ),
)
_fin_call = pl.pallas_call(
    _fin_body,
    out_shape=jax.ShapeDtypeStruct((1, 1), _f32),
)


def kernel(x, edge_index, params):
    info = plsc.get_sparse_core_info()
    nc, ns = info.num_cores, info.num_subcores
    nw, ch, ch_deg, tot_ch, tot_ch_pad, n_pad = _geom(nc, ns)
    degree_call, agg_call = _sc_calls(nc, ns)

    src = edge_index[0]
    dst = edge_index[1]
    pad = tot_ch_pad * CHUNK - E
    ar = jnp.arange(pad, dtype=src.dtype)
    srcp = jnp.concatenate([src, ar % N])
    dstp = jnp.concatenate([dst, N + ar % (n_pad - N)])
    src_w = srcp.reshape(tot_ch_pad, CHUNK)
    dst_w = dstp.reshape(tot_ch_pad, CHUNK)

    deg_pk = degree_call(dst_w)

    p = params
    f32 = jnp.float32

    def pk2(v):
        return jnp.concatenate([v, v], axis=1)

    z = jnp.zeros((H, D_IN), f32)
    menc = jnp.concatenate(
        [jnp.concatenate([p['enc_W'], z], axis=1),
         jnp.concatenate([z, p['enc_W']], axis=1)], axis=0)
    zh = jnp.zeros((H, H), f32)
    mconv = [jnp.concatenate(
        [jnp.concatenate([p['conv_W'][i], zh], axis=1),
         jnp.concatenate([zh, p['conv_W'][i]], axis=1)], axis=0)
        for i in range(L)]
    proll = jnp.roll(jnp.eye(_W2, dtype=f32), H, axis=0)

    eb = pk2(p['enc_b'][None, :])
    cb = [pk2(p['conv_b'][i][None, :]) for i in range(L)]
    bng = [pk2(p['bn_g'][i][None, :]) for i in range(L)]
    bnb = [pk2(p['bn_b'][i][None, :]) for i in range(L)]
    b1 = p['out1_b'][None, :]
    b2 = p['out2_b'][None, :]
    w2 = p['out2_W']

    x_pk = x.reshape(NP, 2 * D_IN)

    dinv, u, s0, m0 = _enc_call(deg_pk, x_pk, menc, eb, mconv[0], proll)
    sums, maxs = [s0], [m0]
    for i in range(L - 1):
        S = agg_call(src_w, dst_w, u.reshape(N, H)).reshape(nc, n_pad // 2, _W2)
        u, si, mi = _mid_call(S, u, dinv, cb[i], bng[i], bnb[i],
                              mconv[i + 1], proll)
        sums.append(si)
        maxs.append(mi)
    S = agg_call(src_w, dst_w, u.reshape(N, H)).reshape(nc, n_pad // 2, _W2)
    ss = jnp.concatenate(sums, axis=0)
    mm = jnp.concatenate(maxs, axis=0)
    K = H * (L + 1)
    W1 = p['out1_W']
    wblocks = ([pk2(W1[:, i * H:(i + 1) * H]) * 0.5 for i in range(L + 1)] +
               [pk2(W1[:, K + i * H:K + (i + 1) * H]) * 0.5
                for i in range(L + 1)] +
               [pk2(W1[:, 2 * K + i * H:2 * K + (i + 1) * H]) * 0.5
                for i in range(L + 1)])
    return _fin_call(S, u, dinv, cb[L - 1], bng[L - 1], bnb[L - 1],
                     ss, mm, proll, *wblocks, b1, w2, b2)
